# baseline (speedup 1.0000x reference)
import functools

import jax
import jax.numpy as jnp
from jax import lax
from jax.experimental import pallas as pl
from jax.experimental.pallas import tpu as pltpu
from jax.experimental.pallas import tpu_sc as plsc

N = 10000
E = 320000
D = 128
G = 64

NC = 2
NS = 16
NW = NC * NS
CHUNK = 32
CPT = 628
EPAD = NS * CPT * CHUNK
NPAD = 10240
NHALF = NPAD // 2
ROWS_PT = NHALF // NS
CW = 16
NBUF = 2
IDXBUF = 4
SSTEPS = CPT // IDXBUF
IGN = -1


@functools.lru_cache(maxsize=None)
def _build_sc():
    mesh = plsc.VectorSubcoreMesh(core_axis_name="c", subcore_axis_name="s",
                                  num_cores=NC, num_subcores=NS)

    @functools.partial(
        pl.kernel,
        mesh=mesh,
        out_type=jax.ShapeDtypeStruct((NPAD, D), jnp.float32),
        scratch_types=[
            pltpu.VMEM((IDXBUF, CHUNK), jnp.int32),
            pltpu.VMEM((IDXBUF, CHUNK), jnp.int32),
            pltpu.VMEM((NBUF * CHUNK, D), jnp.float32),
            pltpu.VMEM_SHARED((NHALF, D), jnp.float32),
            pltpu.VMEM_SHARED((N, D), jnp.float32),
            [pltpu.SemaphoreType.DMA] * IDXBUF,
            [pltpu.SemaphoreType.DMA] * NBUF,
            [pltpu.SemaphoreType.DMA] * NBUF,
        ],
    )
    def agg(src1d, dst1d, xt, zrows, souts, sidx, didx, rows, acc, xs,
            isem, gsem, ssem):
        c = lax.axis_index("c")
        s = lax.axis_index("s")
        r0 = s * ROWS_PT
        base = c * NHALF
        pltpu.sync_copy(zrows.at[pl.ds(0, ROWS_PT)], acc.at[pl.ds(r0, ROWS_PT)])

        @pl.when(s < NS - 1)
        def _():
            pltpu.sync_copy(xt.at[pl.ds(s * 632, 632)],
                            xs.at[pl.ds(s * 632, 632)])

        @pl.when(s == NS - 1)
        def _():
            pltpu.sync_copy(xt.at[pl.ds(9480, 520)], xs.at[pl.ds(9480, 520)])

        e0 = s * (CPT * CHUNK)

        def slot(b):
            return rows.at[pl.ds(b * CHUNK, CHUNK)]

        def idx_load(j, ir):
            pltpu.async_copy(src1d.at[pl.ds(e0 + j * CHUNK, CHUNK)],
                             sidx.at[ir], isem[ir])
            pltpu.async_copy(dst1d.at[pl.ds(e0 + j * CHUNK, CHUNK)],
                             didx.at[ir], isem[ir])

        def idx_wait(j, ir):
            pltpu.make_async_copy(src1d.at[pl.ds(e0 + j * CHUNK, CHUNK)],
                                  sidx.at[ir], isem[ir]).wait()
            pltpu.make_async_copy(dst1d.at[pl.ds(e0 + j * CHUNK, CHUNK)],
                                  didx.at[ir], isem[ir]).wait()

        def transform(ir):
            for k in range(CHUNK // 16):
                dv = didx[ir, pl.ds(k * 16, 16)]
                sv = sidx[ir, pl.ds(k * 16, 16)]
                lv = dv - base
                ok = (lv >= 0) & (lv < NHALF)
                didx[ir, pl.ds(k * 16, 16)] = jnp.where(ok, lv, IGN)
                sidx[ir, pl.ds(k * 16, 16)] = jnp.where(ok, sv, IGN)

        def gref(ir, b):
            return (xs.at[plsc.Indices(sidx.at[ir], ignored_value=IGN)],
                    slot(b))

        def sref(ir, b):
            return (slot(b),
                    acc.at[plsc.Indices(didx.at[ir], ignored_value=IGN)])

        def gather(ir, b):
            g_src, g_dst = gref(ir, b)
            pltpu.async_copy(g_src, g_dst, gsem[b])

        def gather_wait(ir, b):
            g_src, g_dst = gref(ir, b)
            pltpu.make_async_copy(g_src, g_dst, gsem[b]).wait()

        def scatter(ir, b):
            s_src, s_dst = sref(ir, b)
            pltpu.async_copy(s_src, s_dst, ssem[b], add=True)

        def scatter_wait(ir, b):
            s_src, s_dst = sref(ir, b)
            pltpu.make_async_copy(s_src, s_dst, ssem[b]).wait()

        plsc.subcore_barrier()
        for i in range(IDXBUF):
            idx_load(i, i)
        for b in range(NBUF):
            idx_wait(b, b)
            transform(b)
            gather(b, b)

        def superstep(g, carry):
            for i in range(IDXBUF):
                j = g * IDXBUF + i
                b = i % NBUF
                gather_wait(i, b)
                scatter(i, b)
                scatter_wait(i, b)
                idx_load(j + IDXBUF, i)
                i2 = (i + NBUF) % IDXBUF
                idx_wait(j + NBUF, i2)
                transform(i2)
                gather(i2, b)
            return carry

        lax.fori_loop(0, SSTEPS - 1, superstep, 0)
        jt = (SSTEPS - 1) * IDXBUF
        for i in range(IDXBUF):
            b = i % NBUF
            gather_wait(i, b)
            scatter(i, b)
            scatter_wait(i, b)
            if i + NBUF < IDXBUF:
                idx_wait(jt + i + NBUF, i + NBUF)
                transform(i + NBUF)
                gather(i + NBUF, b)
        plsc.subcore_barrier()
        pltpu.sync_copy(acc.at[pl.ds(r0, ROWS_PT)],
                        souts.at[pl.ds(base + r0, ROWS_PT)])

    @functools.partial(
        pl.kernel,
        mesh=mesh,
        out_type=jax.ShapeDtypeStruct((NPAD, D), jnp.float32),
        scratch_types=[
            pltpu.VMEM((IDXBUF, CHUNK), jnp.int32),
            pltpu.VMEM((CHUNK, D), jnp.float32),
            pltpu.VMEM_SHARED((NHALF, D), jnp.float32),
            [pltpu.SemaphoreType.DMA] * IDXBUF,
            [pltpu.SemaphoreType.DMA] * IDXBUF,
        ],
    )
    def cnt(dst1d, zrows, onesc, couts, didx, onesv, cacc, isem, ssem):
        c = lax.axis_index("c")
        s = lax.axis_index("s")
        r0 = s * ROWS_PT
        base = c * NHALF
        pltpu.sync_copy(zrows.at[pl.ds(0, ROWS_PT)],
                        cacc.at[pl.ds(r0, ROWS_PT)])
        pltpu.sync_copy(onesc, onesv)
        e0 = s * (CPT * CHUNK)

        def idx_load(j, ir):
            pltpu.async_copy(dst1d.at[pl.ds(e0 + j * CHUNK, CHUNK)],
                             didx.at[ir], isem[ir])

        def idx_wait(j, ir):
            pltpu.make_async_copy(dst1d.at[pl.ds(e0 + j * CHUNK, CHUNK)],
                                  didx.at[ir], isem[ir]).wait()

        def transform(ir):
            for k in range(CHUNK // 16):
                dv = didx[ir, pl.ds(k * 16, 16)]
                lv = dv - base
                ok = (lv >= 0) & (lv < NHALF)
                didx[ir, pl.ds(k * 16, 16)] = jnp.where(ok, lv, IGN)

        def sref(ir):
            return (onesv,
                    cacc.at[plsc.Indices(didx.at[ir], ignored_value=IGN)])

        def scatter(ir):
            s_src, s_dst = sref(ir)
            pltpu.async_copy(s_src, s_dst, ssem[ir], add=True)

        def scatter_wait(ir):
            s_src, s_dst = sref(ir)
            pltpu.make_async_copy(s_src, s_dst, ssem[ir]).wait()

        plsc.subcore_barrier()
        for i in range(IDXBUF):
            idx_load(i, i)

        def superstep(g, carry):
            for i in range(IDXBUF):
                j = g * IDXBUF + i
                idx_wait(j, i)
                transform(i)
                scatter(i)
                scatter_wait(i)
                idx_load(j + IDXBUF, i)
            return carry

        lax.fori_loop(0, SSTEPS - 1, superstep, 0)
        for i in range(IDXBUF):
            j = (SSTEPS - 1) * IDXBUF + i
            idx_wait(j, i)
            transform(i)
            scatter(i)
            scatter_wait(i)
        plsc.subcore_barrier()
        pltpu.sync_copy(cacc.at[pl.ds(r0, ROWS_PT)],
                        couts.at[pl.ds(base + r0, ROWS_PT)])

    return agg, cnt


def _agg(*args):
    out = _build_sc()[0](*args)
    return out[0] if isinstance(out, (list, tuple)) else out


def _cnt(*args):
    out = _build_sc()[1](*args)
    return out[0] if isinstance(out, (list, tuple)) else out


def _sage_bn(sarr, carr, x, wl, bl, wr, gam, beta):
    ssum = sarr[:N]
    cnt = carr[:N, 0:1]
    mean = ssum / jnp.maximum(cnt, 1.0)
    out = (jnp.dot(mean, wl, preferred_element_type=jnp.float32) + bl
           + jnp.dot(x, wr, preferred_element_type=jnp.float32))
    mu = jnp.mean(out, axis=0, keepdims=True)
    var = jnp.mean((out - mu) ** 2, axis=0, keepdims=True)
    return (out - mu) * lax.rsqrt(var + 1e-5) * gam + beta


def _tc_layer(souts, couts, cur, Wl, bl, Wr, gam, beta):
    def body(s_ref, c_ref, x_ref, wl, blr, wr, gr, br, o_ref):
        o_ref[...] = _sage_bn(s_ref[...], c_ref[...], x_ref[...], wl[...],
                              blr[...], wr[...], gr[...], br[...])

    return pl.pallas_call(
        body, out_shape=jax.ShapeDtypeStruct((N, D), jnp.float32),
    )(souts, couts, cur, Wl, bl, Wr, gam, beta)


def _tc_final(souts, couts, h1, h2, batch_row, Wl, bl, Wr, gam, beta):
    def body(s_ref, c_ref, h1r, h2r, brow, wl, blr, wr, gr, br, o_ref):
        h3 = _sage_bn(s_ref[...], c_ref[...], h2r[...], wl[...], blr[...],
                      wr[...], gr[...], br[...])
        cat = jnp.concatenate([h1r[...], h2r[...], h3], axis=1)
        oh = (lax.broadcasted_iota(jnp.int32, (G, N), 0)
              == brow[...]).astype(jnp.float32)
        gsum = jnp.dot(oh, cat, preferred_element_type=jnp.float32)
        gcnt = jnp.dot(oh, jnp.ones((N, 1), jnp.float32),
                       preferred_element_type=jnp.float32)
        o_ref[...] = gsum / jnp.maximum(gcnt, 1.0)

    return pl.pallas_call(
        body, out_shape=jax.ShapeDtypeStruct((G, 3 * D), jnp.float32),
    )(souts, couts, h1, h2, batch_row, Wl, bl, Wr, gam, beta)


def kernel(x, edge_index, batch, Wl0, bl0, Wr0, gam0, beta0,
           Wl1, bl1, Wr1, gam1, beta1, Wl2, bl2, Wr2, gam2, beta2):
    src = edge_index[0]
    dst = edge_index[1]
    pad = EPAD - E
    src1d = jnp.concatenate([src, jnp.zeros((pad,), jnp.int32)])
    dst1d = jnp.concatenate([dst, jnp.full((pad,), NPAD, jnp.int32)])
    zrows = jnp.zeros((NPAD // NS, D), jnp.float32)
    onesc = jnp.ones((CHUNK, D), jnp.float32)
    batch_row = batch.reshape(1, N)
    row = lambda v: v.reshape(1, D)

    x = x.astype(jnp.float32)
    c1 = _cnt(dst1d, zrows, onesc)[:, :CW]
    c1, src1d, dst1d, x, zrows = lax.optimization_barrier(
        (c1, src1d, dst1d, x, zrows))
    s1 = _agg(src1d, dst1d, x, zrows)
    h1 = _tc_layer(s1, c1, x, Wl0, row(bl0), Wr0, row(gam0), row(beta0))
    s2 = _agg(src1d, dst1d, h1, zrows)
    h2 = _tc_layer(s2, c1, h1, Wl1, row(bl1), Wr1, row(gam1), row(beta1))
    s3 = _agg(src1d, dst1d, h2, zrows)
    return _tc_final(s3, c1, h1, h2, batch_row,
                     Wl2, row(bl2), Wr2, row(gam2), row(beta2))


# speedup vs baseline: 1.0989x; 1.0989x over previous
import functools

import jax
import jax.numpy as jnp
from jax import lax
from jax.experimental import pallas as pl
from jax.experimental.pallas import tpu as pltpu
from jax.experimental.pallas import tpu_sc as plsc

N = 10000
E = 320000
D = 128
G = 64

NC = 2
NS = 16
NW = NC * NS
CHUNK = 32
CPT = 628
EPAD = NS * CPT * CHUNK
NPAD = 10240
NHALF = NPAD // 2
ROWS_PT = NHALF // NS
CW = 16
NBUF = 2
IDXBUF = 4
SSTEPS = CPT // IDXBUF
IGN = -1
CCH = 64
CCPT = 160
CEPAD = NW * CCPT * CCH
NDUMMY = 512
CACC_ROWS = NPAD + NDUMMY


@functools.lru_cache(maxsize=None)
def _build_sc():
    mesh = plsc.VectorSubcoreMesh(core_axis_name="c", subcore_axis_name="s",
                                  num_cores=NC, num_subcores=NS)

    @functools.partial(
        pl.kernel,
        mesh=mesh,
        out_type=jax.ShapeDtypeStruct((NPAD, D), jnp.float32),
        scratch_types=[
            pltpu.VMEM((IDXBUF, CHUNK), jnp.int32),
            pltpu.VMEM((IDXBUF, CHUNK), jnp.int32),
            pltpu.VMEM((NBUF * CHUNK, D), jnp.float32),
            pltpu.VMEM_SHARED((NHALF, D), jnp.float32),
            pltpu.VMEM_SHARED((N, D), jnp.float32),
            [pltpu.SemaphoreType.DMA] * IDXBUF,
            [pltpu.SemaphoreType.DMA] * NBUF,
            [pltpu.SemaphoreType.DMA] * NBUF,
        ],
    )
    def agg(src1d, dst1d, xt, zrows, souts, sidx, didx, rows, acc, xs,
            isem, gsem, ssem):
        c = lax.axis_index("c")
        s = lax.axis_index("s")
        r0 = s * ROWS_PT
        base = c * NHALF
        pltpu.sync_copy(zrows.at[pl.ds(0, ROWS_PT)], acc.at[pl.ds(r0, ROWS_PT)])

        @pl.when(s < NS - 1)
        def _():
            pltpu.sync_copy(xt.at[pl.ds(s * 632, 632)],
                            xs.at[pl.ds(s * 632, 632)])

        @pl.when(s == NS - 1)
        def _():
            pltpu.sync_copy(xt.at[pl.ds(9480, 520)], xs.at[pl.ds(9480, 520)])

        e0 = s * (CPT * CHUNK)

        def slot(b):
            return rows.at[pl.ds(b * CHUNK, CHUNK)]

        def idx_load(j, ir):
            pltpu.async_copy(src1d.at[pl.ds(e0 + j * CHUNK, CHUNK)],
                             sidx.at[ir], isem[ir])
            pltpu.async_copy(dst1d.at[pl.ds(e0 + j * CHUNK, CHUNK)],
                             didx.at[ir], isem[ir])

        def idx_wait(j, ir):
            pltpu.make_async_copy(src1d.at[pl.ds(e0 + j * CHUNK, CHUNK)],
                                  sidx.at[ir], isem[ir]).wait()
            pltpu.make_async_copy(dst1d.at[pl.ds(e0 + j * CHUNK, CHUNK)],
                                  didx.at[ir], isem[ir]).wait()

        def transform(ir):
            for k in range(CHUNK // 16):
                dv = didx[ir, pl.ds(k * 16, 16)]
                sv = sidx[ir, pl.ds(k * 16, 16)]
                lv = dv - base
                ok = (lv >= 0) & (lv < NHALF)
                didx[ir, pl.ds(k * 16, 16)] = jnp.where(ok, lv, IGN)
                sidx[ir, pl.ds(k * 16, 16)] = jnp.where(ok, sv, IGN)

        def gref(ir, b):
            return (xs.at[plsc.Indices(sidx.at[ir], ignored_value=IGN)],
                    slot(b))

        def sref(ir, b):
            return (slot(b),
                    acc.at[plsc.Indices(didx.at[ir], ignored_value=IGN)])

        def gather(ir, b):
            g_src, g_dst = gref(ir, b)
            pltpu.async_copy(g_src, g_dst, gsem[b])

        def gather_wait(ir, b):
            g_src, g_dst = gref(ir, b)
            pltpu.make_async_copy(g_src, g_dst, gsem[b]).wait()

        def scatter(ir, b):
            s_src, s_dst = sref(ir, b)
            pltpu.async_copy(s_src, s_dst, ssem[b], add=True)

        def scatter_wait(ir, b):
            s_src, s_dst = sref(ir, b)
            pltpu.make_async_copy(s_src, s_dst, ssem[b]).wait()

        plsc.subcore_barrier()
        for i in range(IDXBUF):
            idx_load(i, i)
        for b in range(NBUF):
            idx_wait(b, b)
            transform(b)
            gather(b, b)

        def superstep(g, carry):
            for i in range(IDXBUF):
                j = g * IDXBUF + i
                b = i % NBUF
                gather_wait(i, b)
                scatter(i, b)
                scatter_wait(i, b)
                idx_load(j + IDXBUF, i)
                i2 = (i + NBUF) % IDXBUF
                idx_wait(j + NBUF, i2)
                transform(i2)
                gather(i2, b)
            return carry

        lax.fori_loop(0, SSTEPS - 1, superstep, 0)
        jt = (SSTEPS - 1) * IDXBUF
        for i in range(IDXBUF):
            b = i % NBUF
            gather_wait(i, b)
            scatter(i, b)
            scatter_wait(i, b)
            if i + NBUF < IDXBUF:
                idx_wait(jt + i + NBUF, i + NBUF)
                transform(i + NBUF)
                gather(i + NBUF, b)
        plsc.subcore_barrier()
        pltpu.sync_copy(acc.at[pl.ds(r0, ROWS_PT)],
                        souts.at[pl.ds(base + r0, ROWS_PT)])

    @functools.partial(
        pl.kernel,
        mesh=mesh,
        out_type=jax.ShapeDtypeStruct((2 * NPAD, D), jnp.float32),
        scratch_types=[
            pltpu.VMEM((CCPT, CCH), jnp.int32),
            pltpu.VMEM((CCH, D), jnp.float32),
            pltpu.VMEM_SHARED((CACC_ROWS, D), jnp.float32),
        ],
    )
    def cnt(dst2d, zrows, onesc, couts, dstbuf, onesv, cacc):
        c = lax.axis_index("c")
        s = lax.axis_index("s")
        wid = s * NC + c
        r0 = s * (NPAD // NS)
        pltpu.sync_copy(zrows, cacc.at[pl.ds(r0, NPAD // NS)])
        pltpu.sync_copy(onesc, onesv)
        ch0 = wid * CCPT
        pltpu.sync_copy(dst2d.at[pl.ds(ch0, CCPT)], dstbuf)
        plsc.subcore_barrier()

        def step(j, carry):
            pltpu.sync_copy(onesv, cacc.at[dstbuf.at[j]], add=True)
            return carry

        lax.fori_loop(0, CCPT, step, 0)
        plsc.subcore_barrier()
        o0 = c * NPAD + r0
        pltpu.sync_copy(cacc.at[pl.ds(r0, NPAD // NS)],
                        couts.at[pl.ds(o0, NPAD // NS)])

    return agg, cnt


def _agg(*args):
    out = _build_sc()[0](*args)
    return out[0] if isinstance(out, (list, tuple)) else out


def _cnt(*args):
    out = _build_sc()[1](*args)
    return out[0] if isinstance(out, (list, tuple)) else out


def _sage_bn(sarr, carr, x, wl, bl, wr, gam, beta):
    ssum = sarr[:N]
    cnt = carr[:N, 0:1] + carr[NPAD:NPAD + N, 0:1]
    mean = ssum / jnp.maximum(cnt, 1.0)
    out = (jnp.dot(mean, wl, preferred_element_type=jnp.float32) + bl
           + jnp.dot(x, wr, preferred_element_type=jnp.float32))
    mu = jnp.mean(out, axis=0, keepdims=True)
    var = jnp.mean((out - mu) ** 2, axis=0, keepdims=True)
    return (out - mu) * lax.rsqrt(var + 1e-5) * gam + beta


def _tc_layer(souts, couts, cur, Wl, bl, Wr, gam, beta):
    def body(s_ref, c_ref, x_ref, wl, blr, wr, gr, br, o_ref):
        o_ref[...] = _sage_bn(s_ref[...], c_ref[...], x_ref[...], wl[...],
                              blr[...], wr[...], gr[...], br[...])

    return pl.pallas_call(
        body, out_shape=jax.ShapeDtypeStruct((N, D), jnp.float32),
    )(souts, couts, cur, Wl, bl, Wr, gam, beta)


def _tc_final(souts, couts, h1, h2, batch_row, Wl, bl, Wr, gam, beta):
    def body(s_ref, c_ref, h1r, h2r, brow, wl, blr, wr, gr, br, o_ref):
        h3 = _sage_bn(s_ref[...], c_ref[...], h2r[...], wl[...], blr[...],
                      wr[...], gr[...], br[...])
        cat = jnp.concatenate([h1r[...], h2r[...], h3], axis=1)
        oh = (lax.broadcasted_iota(jnp.int32, (G, N), 0)
              == brow[...]).astype(jnp.float32)
        gsum = jnp.dot(oh, cat, preferred_element_type=jnp.float32)
        gcnt = jnp.dot(oh, jnp.ones((N, 1), jnp.float32),
                       preferred_element_type=jnp.float32)
        o_ref[...] = gsum / jnp.maximum(gcnt, 1.0)

    return pl.pallas_call(
        body, out_shape=jax.ShapeDtypeStruct((G, 3 * D), jnp.float32),
    )(souts, couts, h1, h2, batch_row, Wl, bl, Wr, gam, beta)


def kernel(x, edge_index, batch, Wl0, bl0, Wr0, gam0, beta0,
           Wl1, bl1, Wr1, gam1, beta1, Wl2, bl2, Wr2, gam2, beta2):
    src = edge_index[0]
    dst = edge_index[1]
    pad = EPAD - E
    src1d = jnp.concatenate([src, jnp.zeros((pad,), jnp.int32)])
    dst1d = jnp.concatenate([dst, jnp.full((pad,), NPAD, jnp.int32)])
    cpad = CEPAD - E
    dst2d = jnp.concatenate(
        [dst, NPAD + (jnp.arange(cpad, dtype=jnp.int32) % NDUMMY)]
    ).reshape(NW * CCPT, CCH)
    zrows = jnp.zeros((NPAD // NS, D), jnp.float32)
    onesc = jnp.ones((CCH, D), jnp.float32)
    batch_row = batch.reshape(1, N)
    row = lambda v: v.reshape(1, D)

    x = x.astype(jnp.float32)
    c1 = _cnt(dst2d, zrows, onesc)[:, :CW]
    c1, src1d, dst1d, x, zrows = lax.optimization_barrier(
        (c1, src1d, dst1d, x, zrows))
    s1 = _agg(src1d, dst1d, x, zrows)
    h1 = _tc_layer(s1, c1, x, Wl0, row(bl0), Wr0, row(gam0), row(beta0))
    s2 = _agg(src1d, dst1d, h1, zrows)
    h2 = _tc_layer(s2, c1, h1, Wl1, row(bl1), Wr1, row(gam1), row(beta1))
    s3 = _agg(src1d, dst1d, h2, zrows)
    return _tc_final(s3, c1, h1, h2, batch_row,
                     Wl2, row(bl2), Wr2, row(gam2), row(beta2))
